# baseline (speedup 1.0000x reference)
import functools

import jax
import jax.numpy as jnp
from jax import lax
from jax.experimental import pallas as pl
from jax.experimental.pallas import tpu as pltpu
from jax.experimental.pallas import tpu_sc as plsc

_B, _N, _CIN, _COUT, _MID, _K = 8, 2048, 64, 64, 64, 16
_QT = 512
_QA = 256
_NW = 32
_CH = 128
_NCHUNK = (_B * _N * _K) // (_NW * _CH)
_RING = 2


def _proj_knn_body(x_ref, p1_ref, p1t_ref, wpwc_ref, wl1_ref, b1_ref,
                   wq_ref, bq_ref, wk_ref, bk_ref, wv_ref, bv_ref, wd1_ref,
                   ft_ref, q_ref, kv_ref, bt_ref, idx_ref):
    b = pl.program_id(0)
    xb = x_ref[0]
    f = jnp.dot(wpwc_ref[...], xb, preferred_element_type=jnp.float32)
    ft_ref[0] = f
    h = jnp.dot(wl1_ref[...], f, preferred_element_type=jnp.float32) + b1_ref[...]
    qrows = lax.dot_general(h, wq_ref[...], (((0,), (1,)), ((), ())),
                            preferred_element_type=jnp.float32) + bq_ref[...]
    krows = lax.dot_general(h, wk_ref[...], (((0,), (1,)), ((), ())),
                            preferred_element_type=jnp.float32) + bk_ref[...]
    vrows = lax.dot_general(h, wv_ref[...], (((0,), (1,)), ((), ())),
                            preferred_element_type=jnp.float32) + bv_ref[...]
    q_ref[0] = qrows
    kv_ref[0] = jnp.concatenate([krows, vrows], axis=1)
    bt = lax.dot_general(p1_ref[0], wd1_ref[...], (((1,), (1,)), ((), ())),
                         preferred_element_type=jnp.float32)
    bt_ref[0] = jnp.concatenate([bt, jnp.zeros_like(bt)], axis=1)

    qx = p1_ref[0, :, 0:1]
    qy = p1_ref[0, :, 1:2]
    qz = p1_ref[0, :, 2:3]
    px = p1t_ref[0, 0:1, :]
    py = p1t_ref[0, 1:2, :]
    pz = p1t_ref[0, 2:3, :]
    d2 = (qx - px) ** 2 + (qy - py) ** 2 + (qz - pz) ** 2
    col = lax.broadcasted_iota(jnp.int32, (_QT, _N), 1)
    base = b * _N
    cols = []
    for _ in range(_K):
        am = jnp.argmin(d2, axis=1).astype(jnp.int32)[:, None]
        cols.append(am)
        d2 = jnp.where(col == am, jnp.float32(1e30), d2)
    idx_ref[0] = jnp.concatenate(cols, axis=1) + base


def _proj_knn(x, p1, p1t, wpwc, wl1, b1c, wq, bqr, wk, bkr, wv, bvr, wd1):
    nb = x.shape[0]
    nt = _N // _QT
    full = lambda s: pl.BlockSpec(s, lambda b, t: tuple(0 for _ in s))
    return pl.pallas_call(
        _proj_knn_body,
        grid=(nb, nt),
        in_specs=[
            pl.BlockSpec((1, _CIN, _QT), lambda b, t: (b, 0, t)),
            pl.BlockSpec((1, _QT, 3), lambda b, t: (b, t, 0)),
            pl.BlockSpec((1, 3, _N), lambda b, t: (b, 0, 0)),
            full((_COUT, _CIN)),
            full((_MID, _COUT)),
            full((_MID, 1)),
            full((_MID, _MID)), full((1, _MID)),
            full((_MID, _MID)), full((1, _MID)),
            full((_MID, _MID)), full((1, _MID)),
            full((_MID, 3)),
        ],
        out_specs=[
            pl.BlockSpec((1, _COUT, _QT), lambda b, t: (b, 0, t)),
            pl.BlockSpec((1, _QT, _MID), lambda b, t: (b, t, 0)),
            pl.BlockSpec((1, _QT, 2 * _MID), lambda b, t: (b, t, 0)),
            pl.BlockSpec((1, _QT, 2 * _MID), lambda b, t: (b, t, 0)),
            pl.BlockSpec((1, _QT, _K), lambda b, t: (b, t, 0)),
        ],
        out_shape=[
            jax.ShapeDtypeStruct((nb, _COUT, _N), jnp.float32),
            jax.ShapeDtypeStruct((nb, _N, _MID), jnp.float32),
            jax.ShapeDtypeStruct((nb, _N, 2 * _MID), jnp.float32),
            jax.ShapeDtypeStruct((nb, _N, 2 * _MID), jnp.float32),
            jax.ShapeDtypeStruct((nb, _N, _K), jnp.int32),
        ],
    )(x, p1, p1t, wpwc, wl1, b1c, wq, bqr, wk, bkr, wv, bvr, wd1)


def _sc_gather_body(nchunk, idx_hbm, kv_hbm, bt_hbm, kvj_hbm, btj_hbm,
                    idx_v, bk0, bk1, bb0, bb1, sg0, sg1, sw0, sw1):
    wid = lax.axis_index("s") * 2 + lax.axis_index("c")
    bks = [bk0, bk1]
    bbs = [bb0, bb1]
    sgs = [sg0, sg1]
    sws = [sw0, sw1]
    rows_per_w = nchunk * _CH

    pltpu.sync_copy(idx_hbm.at[wid], idx_v)

    def fire_gather(c, r):
        pltpu.async_copy(kv_hbm.at[idx_v.at[c]], bks[r], sgs[r])
        pltpu.async_copy(bt_hbm.at[idx_v.at[c]], bbs[r], sgs[r])

    def wait_gather(r):
        pltpu.make_async_copy(kv_hbm.at[pl.ds(0, _CH)], bks[r], sgs[r]).wait()
        pltpu.make_async_copy(bt_hbm.at[pl.ds(0, _CH)], bbs[r], sgs[r]).wait()

    def fire_write(c, r):
        row0 = wid * rows_per_w + c * _CH
        pltpu.async_copy(bks[r], kvj_hbm.at[pl.ds(row0, _CH)], sws[r])
        pltpu.async_copy(bbs[r], btj_hbm.at[pl.ds(row0, _CH)], sws[r])

    def wait_write(r):
        pltpu.make_async_copy(kvj_hbm.at[pl.ds(0, _CH)], bks[r], sws[r]).wait()
        pltpu.make_async_copy(btj_hbm.at[pl.ds(0, _CH)], bbs[r], sws[r]).wait()

    for r in range(_RING):
        fire_gather(r, r)

    def group(g, carry):
        for r in range(_RING):
            c = g * _RING + r
            wait_gather(r)
            fire_write(c, r)
        for r in range(_RING):
            c2 = g * _RING + r + _RING

            @pl.when(c2 < nchunk)
            def _():
                wait_write(r)
                fire_gather(c2, r)
        return carry

    lax.fori_loop(0, nchunk // _RING, group, 0)
    for r in range(_RING):
        wait_write(r)


def _sc_gather(idx_r, kvf, btf):
    g = kvf.shape[0] * _K
    nchunk = g // (_NW * _CH)
    mesh = plsc.VectorSubcoreMesh(core_axis_name="c", subcore_axis_name="s")
    kfn = functools.partial(
        pl.kernel,
        mesh=mesh,
        compiler_params=pltpu.CompilerParams(use_tc_tiling_on_sc=True),
        out_type=[
            jax.ShapeDtypeStruct((g, 2 * _MID), jnp.float32),
            jax.ShapeDtypeStruct((g, 2 * _MID), jnp.float32),
        ],
        scratch_types=(
            [pltpu.VMEM((nchunk, _CH), jnp.int32)]
            + [pltpu.VMEM((_CH, 2 * _MID), jnp.float32) for _ in range(2 * _RING)]
            + [pltpu.SemaphoreType.DMA for _ in range(2 * _RING)]
        ),
    )(functools.partial(_sc_gather_body, nchunk))
    return kfn(idx_r, kvf, btf)


def _attn_body(q_ref, p1_ref, ft_ref, kvj_ref, btj_ref,
               wd1_ref, bd1_ref, wd2_ref, bd2_ref,
               wg1_ref, bg1_ref, wg2_ref, bg2_ref,
               wl2_ref, bl2_ref, y_ref):
    qk = _QA * _K
    p1b = p1_ref[0]
    a_q = lax.dot_general(p1b, wd1_ref[...], (((1,), (1,)), ((), ())),
                          preferred_element_type=jnp.float32) + bd1_ref[...]
    rel1 = (jnp.reshape(a_q, (_QA, 1, _MID))
            - jnp.reshape(btj_ref[:, 0:_MID], (_QA, _K, _MID)))
    relu1 = jnp.maximum(rel1, 0.0)
    pos = lax.dot_general(jnp.reshape(relu1, (qk, _MID)), wd2_ref[...],
                          (((1,), (1,)), ((), ())),
                          preferred_element_type=jnp.float32) + bd2_ref[...]
    pos3 = jnp.reshape(pos, (_QA, _K, _MID))

    e3 = (jnp.reshape(q_ref[0], (_QA, 1, _MID))
          - jnp.reshape(kvj_ref[:, 0:_MID], (_QA, _K, _MID))
          + pos3)
    t1 = jnp.maximum(
        lax.dot_general(jnp.reshape(e3, (qk, _MID)), wg1_ref[...],
                        (((1,), (1,)), ((), ())),
                        preferred_element_type=jnp.float32) + bg1_ref[...], 0.0)
    logits = lax.dot_general(t1, wg2_ref[...], (((1,), (1,)), ((), ())),
                             preferred_element_type=jnp.float32) + bg2_ref[...]
    l3 = jnp.reshape(logits, (_QA, _K, _MID))
    mx = jnp.max(l3, axis=1, keepdims=True)
    ex = jnp.exp(l3 - mx)
    attn = ex * (1.0 / jnp.sum(ex, axis=1, keepdims=True))

    vp = jnp.reshape(kvj_ref[:, _MID:2 * _MID], (_QA, _K, _MID)) + pos3
    agg = jnp.sum(attn * vp, axis=1)
    yt = lax.dot_general(wl2_ref[...], agg, (((1,), (1,)), ((), ())),
                         preferred_element_type=jnp.float32)
    y_ref[0] = yt + bl2_ref[...] + ft_ref[0]


def _attn(q, p1, ft, kvj, btj, wd1, bd1r, wd2, bd2r, wg1, bg1r, wg2, bg2r,
          wl2, bl2c):
    nb = q.shape[0]
    nt = _N // _QA
    full = lambda s: pl.BlockSpec(s, lambda b, t: tuple(0 for _ in s))
    return pl.pallas_call(
        _attn_body,
        grid=(nb, nt),
        in_specs=[
            pl.BlockSpec((1, _QA, _MID), lambda b, t: (b, t, 0)),
            pl.BlockSpec((1, _QA, 3), lambda b, t: (b, t, 0)),
            pl.BlockSpec((1, _COUT, _QA), lambda b, t: (b, 0, t)),
            pl.BlockSpec((_QA * _K, 2 * _MID), lambda b, t: (b * (_N // _QA) + t, 0)),
            pl.BlockSpec((_QA * _K, 2 * _MID), lambda b, t: (b * (_N // _QA) + t, 0)),
            full((_MID, 3)), full((1, _MID)),
            full((_MID, _MID)), full((1, _MID)),
            full((_MID, _MID)), full((1, _MID)),
            full((_MID, _MID)), full((1, _MID)),
            full((_COUT, _MID)), full((_COUT, 1)),
        ],
        out_specs=pl.BlockSpec((1, _COUT, _QA), lambda b, t: (b, 0, t)),
        out_shape=jax.ShapeDtypeStruct((nb, _COUT, _N), jnp.float32),
    )(q, p1, ft, kvj, btj, wd1, bd1r, wd2, bd2r, wg1, bg1r, wg2, bg2r,
      wl2, bl2c)


def kernel(x, p1, W_pwc, W_lin1, b_lin1, Wq, bq, Wk, bk, Wv, bv,
           Wd1, bd1, Wd2, bd2, Wg1, bg1, Wg2, bg2, W_lin2, b_lin2):
    p1t = jnp.transpose(p1, (0, 2, 1))
    hb = _B // 2

    proj = [
        _proj_knn(
            x[s:s + hb], p1[s:s + hb], p1t[s:s + hb],
            W_pwc, W_lin1, b_lin1.reshape(_MID, 1),
            Wq, bq.reshape(1, _MID), Wk, bk.reshape(1, _MID),
            Wv, bv.reshape(1, _MID), Wd1)
        for s in (0, hb)
    ]
    gath = [
        _sc_gather(idxg.reshape(_NW, -1, _CH),
                   kv.reshape(hb * _N, 2 * _MID),
                   bt.reshape(hb * _N, 2 * _MID))
        for (ft, q, kv, bt, idxg) in proj
    ]
    ys = [
        _attn(q, p1[s:s + hb], ft, kvj, btj,
              Wd1, bd1.reshape(1, _MID), Wd2, bd2.reshape(1, _MID),
              Wg1, bg1.reshape(1, _MID), Wg2, bg2.reshape(1, _MID),
              W_lin2, b_lin2.reshape(_COUT, 1))
        for s, (ft, q, kv, bt, idxg), (kvj, btj) in zip((0, hb), proj, gath)
    ]
    return (jnp.concatenate(ys, axis=0), p1)


# speedup vs baseline: 1.0182x; 1.0182x over previous
import functools

import jax
import jax.numpy as jnp
from jax import lax
from jax.experimental import pallas as pl
from jax.experimental.pallas import tpu as pltpu
from jax.experimental.pallas import tpu_sc as plsc

_B, _N, _CIN, _COUT, _MID, _K = 8, 2048, 64, 64, 64, 16
_QT = 512
_QA = 256
_NW = 32
_CH = 128
_NCHUNK = (_B * _N * _K) // (_NW * _CH)
_RING = 2


def _proj_knn_body(x_ref, p1_ref, p1t_ref, wpwc_ref, wl1_ref, b1_ref,
                   wq_ref, bq_ref, wk_ref, bk_ref, wv_ref, bv_ref, wd1_ref,
                   ft_ref, q_ref, kv_ref, bt_ref, idx_ref):
    b = pl.program_id(0)
    xb = x_ref[0]
    f = jnp.dot(wpwc_ref[...], xb, preferred_element_type=jnp.float32)
    ft_ref[0] = f
    h = jnp.dot(wl1_ref[...], f, preferred_element_type=jnp.float32) + b1_ref[...]
    qrows = lax.dot_general(h, wq_ref[...], (((0,), (1,)), ((), ())),
                            preferred_element_type=jnp.float32) + bq_ref[...]
    krows = lax.dot_general(h, wk_ref[...], (((0,), (1,)), ((), ())),
                            preferred_element_type=jnp.float32) + bk_ref[...]
    vrows = lax.dot_general(h, wv_ref[...], (((0,), (1,)), ((), ())),
                            preferred_element_type=jnp.float32) + bv_ref[...]
    q_ref[0] = qrows
    kv_ref[0] = jnp.concatenate([krows, vrows], axis=1)
    bt = lax.dot_general(p1_ref[0], wd1_ref[...], (((1,), (1,)), ((), ())),
                         preferred_element_type=jnp.float32)
    bt_ref[0] = jnp.concatenate([bt, jnp.zeros_like(bt)], axis=1)

    qx = p1_ref[0, :, 0:1]
    qy = p1_ref[0, :, 1:2]
    qz = p1_ref[0, :, 2:3]
    px = p1t_ref[0, 0:1, :]
    py = p1t_ref[0, 1:2, :]
    pz = p1t_ref[0, 2:3, :]
    d2 = (qx - px) ** 2 + (qy - py) ** 2 + (qz - pz) ** 2
    col = lax.broadcasted_iota(jnp.int32, (_QT, _N), 1)
    base = b * _N
    cols = []
    for _ in range(_K):
        am = jnp.argmin(d2, axis=1).astype(jnp.int32)[:, None]
        cols.append(am)
        d2 = jnp.where(col == am, jnp.float32(1e30), d2)
    idx_ref[0] = jnp.concatenate(cols, axis=1) + base


def _proj_knn(x, p1, p1t, wpwc, wl1, b1c, wq, bqr, wk, bkr, wv, bvr, wd1):
    nb = x.shape[0]
    nt = _N // _QT
    full = lambda s: pl.BlockSpec(s, lambda b, t: tuple(0 for _ in s))
    return pl.pallas_call(
        _proj_knn_body,
        grid=(nb, nt),
        in_specs=[
            pl.BlockSpec((1, _CIN, _QT), lambda b, t: (b, 0, t)),
            pl.BlockSpec((1, _QT, 3), lambda b, t: (b, t, 0)),
            pl.BlockSpec((1, 3, _N), lambda b, t: (b, 0, 0)),
            full((_COUT, _CIN)),
            full((_MID, _COUT)),
            full((_MID, 1)),
            full((_MID, _MID)), full((1, _MID)),
            full((_MID, _MID)), full((1, _MID)),
            full((_MID, _MID)), full((1, _MID)),
            full((_MID, 3)),
        ],
        out_specs=[
            pl.BlockSpec((1, _COUT, _QT), lambda b, t: (b, 0, t)),
            pl.BlockSpec((1, _QT, _MID), lambda b, t: (b, t, 0)),
            pl.BlockSpec((1, _QT, 2 * _MID), lambda b, t: (b, t, 0)),
            pl.BlockSpec((1, _QT, 2 * _MID), lambda b, t: (b, t, 0)),
            pl.BlockSpec((1, _QT, _K), lambda b, t: (b, t, 0)),
        ],
        out_shape=[
            jax.ShapeDtypeStruct((nb, _COUT, _N), jnp.float32),
            jax.ShapeDtypeStruct((nb, _N, _MID), jnp.float32),
            jax.ShapeDtypeStruct((nb, _N, 2 * _MID), jnp.float32),
            jax.ShapeDtypeStruct((nb, _N, 2 * _MID), jnp.float32),
            jax.ShapeDtypeStruct((nb, _N, _K), jnp.int32),
        ],
    )(x, p1, p1t, wpwc, wl1, b1c, wq, bqr, wk, bkr, wv, bvr, wd1)


def _sc_gather_body(nchunk, idx_hbm, kv_hbm, bt_hbm, kvj_hbm, btj_hbm,
                    idx_v, bk0, bk1, bb0, bb1, sg0, sg1, sw0, sw1):
    wid = lax.axis_index("s") * 2 + lax.axis_index("c")
    bks = [bk0, bk1]
    bbs = [bb0, bb1]
    sgs = [sg0, sg1]
    sws = [sw0, sw1]
    rows_per_w = nchunk * _CH

    pltpu.sync_copy(idx_hbm.at[wid], idx_v)

    def fire_gather(c, r):
        pltpu.async_copy(kv_hbm.at[idx_v.at[c]], bks[r], sgs[r])
        pltpu.async_copy(bt_hbm.at[idx_v.at[c]], bbs[r], sgs[r])

    def wait_gather(r):
        pltpu.make_async_copy(kv_hbm.at[pl.ds(0, _CH)], bks[r], sgs[r]).wait()
        pltpu.make_async_copy(bt_hbm.at[pl.ds(0, _CH)], bbs[r], sgs[r]).wait()

    def fire_write(c, r):
        row0 = wid * rows_per_w + c * _CH
        pltpu.async_copy(bks[r], kvj_hbm.at[pl.ds(row0, _CH)], sws[r])
        pltpu.async_copy(bbs[r], btj_hbm.at[pl.ds(row0, _CH)], sws[r])

    def wait_write(r):
        pltpu.make_async_copy(kvj_hbm.at[pl.ds(0, _CH)], bks[r], sws[r]).wait()
        pltpu.make_async_copy(btj_hbm.at[pl.ds(0, _CH)], bbs[r], sws[r]).wait()

    for r in range(_RING):
        fire_gather(r, r)

    def group(g, carry):
        for r in range(_RING):
            c = g * _RING + r
            wait_gather(r)
            fire_write(c, r)
        for r in range(_RING):
            c2 = g * _RING + r + _RING

            @pl.when(c2 < nchunk)
            def _():
                wait_write(r)
                fire_gather(c2, r)
        return carry

    lax.fori_loop(0, nchunk // _RING, group, 0)
    for r in range(_RING):
        wait_write(r)


def _sc_gather(idx_r, kvf, btf):
    g = kvf.shape[0] * _K
    nchunk = g // (_NW * _CH)
    mesh = plsc.VectorSubcoreMesh(core_axis_name="c", subcore_axis_name="s")
    kfn = functools.partial(
        pl.kernel,
        mesh=mesh,
        compiler_params=pltpu.CompilerParams(use_tc_tiling_on_sc=True),
        out_type=[
            jax.ShapeDtypeStruct((g, 2 * _MID), jnp.float32),
            jax.ShapeDtypeStruct((g, 2 * _MID), jnp.float32),
        ],
        scratch_types=(
            [pltpu.VMEM((nchunk, _CH), jnp.int32)]
            + [pltpu.VMEM((_CH, 2 * _MID), jnp.float32) for _ in range(2 * _RING)]
            + [pltpu.SemaphoreType.DMA for _ in range(2 * _RING)]
        ),
    )(functools.partial(_sc_gather_body, nchunk))
    return kfn(idx_r, kvf, btf)


def _attn_body(q_ref, p1_ref, ft_ref, kvj_ref, btj_ref,
               wd1_ref, bd1_ref, wd2_ref, bd2_ref,
               wg1_ref, bg1_ref, wg2_ref, bg2_ref,
               wl2_ref, bl2_ref, y_ref):
    qk = _QA * _K
    p1b = p1_ref[0]
    a_q = lax.dot_general(p1b, wd1_ref[...], (((1,), (1,)), ((), ())),
                          preferred_element_type=jnp.float32) + bd1_ref[...]
    rel1 = (jnp.reshape(a_q, (1, _QA, _MID))
            - btj_ref[:, :, 0:_MID])
    relu1 = jnp.maximum(rel1, 0.0)
    pos = lax.dot_general(jnp.reshape(relu1, (qk, _MID)), wd2_ref[...],
                          (((1,), (1,)), ((), ())),
                          preferred_element_type=jnp.float32) + bd2_ref[...]
    pos3 = jnp.reshape(pos, (_K, _QA, _MID))

    e3 = (jnp.reshape(q_ref[0], (1, _QA, _MID))
          - kvj_ref[:, :, 0:_MID]
          + pos3)
    t1 = jnp.maximum(
        lax.dot_general(jnp.reshape(e3, (qk, _MID)), wg1_ref[...],
                        (((1,), (1,)), ((), ())),
                        preferred_element_type=jnp.float32) + bg1_ref[...], 0.0)
    logits = lax.dot_general(t1, wg2_ref[...], (((1,), (1,)), ((), ())),
                             preferred_element_type=jnp.float32) + bg2_ref[...]
    l3 = jnp.reshape(logits, (_K, _QA, _MID))
    mx = jnp.max(l3, axis=0, keepdims=True)
    ex = jnp.exp(l3 - mx)
    attn = ex * (1.0 / jnp.sum(ex, axis=0, keepdims=True))

    vp = kvj_ref[:, :, _MID:2 * _MID] + pos3
    agg = jnp.sum(attn * vp, axis=0)
    yt = lax.dot_general(wl2_ref[...], agg, (((1,), (1,)), ((), ())),
                         preferred_element_type=jnp.float32)
    y_ref[0] = yt + bl2_ref[...] + ft_ref[0]


def _attn(q, p1, ft, kvj, btj, wd1, bd1r, wd2, bd2r, wg1, bg1r, wg2, bg2r,
          wl2, bl2c):
    nb = q.shape[0]
    nt = _N // _QA
    full = lambda s: pl.BlockSpec(s, lambda b, t: tuple(0 for _ in s))
    return pl.pallas_call(
        _attn_body,
        grid=(nb, nt),
        in_specs=[
            pl.BlockSpec((1, _QA, _MID), lambda b, t: (b, t, 0)),
            pl.BlockSpec((1, _QA, 3), lambda b, t: (b, t, 0)),
            pl.BlockSpec((1, _COUT, _QA), lambda b, t: (b, 0, t)),
            pl.BlockSpec((_K, _QA, 2 * _MID), lambda b, t: (b, t, 0)),
            pl.BlockSpec((_K, _QA, 2 * _MID), lambda b, t: (b, t, 0)),
            full((_MID, 3)), full((1, _MID)),
            full((_MID, _MID)), full((1, _MID)),
            full((_MID, _MID)), full((1, _MID)),
            full((_MID, _MID)), full((1, _MID)),
            full((_COUT, _MID)), full((_COUT, 1)),
        ],
        out_specs=pl.BlockSpec((1, _COUT, _QA), lambda b, t: (b, 0, t)),
        out_shape=jax.ShapeDtypeStruct((nb, _COUT, _N), jnp.float32),
    )(q, p1, ft, kvj, btj, wd1, bd1r, wd2, bd2r, wg1, bg1r, wg2, bg2r,
      wl2, bl2c)


def kernel(x, p1, W_pwc, W_lin1, b_lin1, Wq, bq, Wk, bk, Wv, bv,
           Wd1, bd1, Wd2, bd2, Wg1, bg1, Wg2, bg2, W_lin2, b_lin2):
    p1t = jnp.transpose(p1, (0, 2, 1))
    hb = _B // 2

    proj = [
        _proj_knn(
            x[s:s + hb], p1[s:s + hb], p1t[s:s + hb],
            W_pwc, W_lin1, b_lin1.reshape(_MID, 1),
            Wq, bq.reshape(1, _MID), Wk, bk.reshape(1, _MID),
            Wv, bv.reshape(1, _MID), Wd1)
        for s in (0, hb)
    ]
    gath = [
        _sc_gather(jnp.transpose(idxg, (0, 2, 1)).reshape(_NW, -1, _CH),
                   kv.reshape(hb * _N, 2 * _MID),
                   bt.reshape(hb * _N, 2 * _MID))
        for (ft, q, kv, bt, idxg) in proj
    ]
    ys = [
        _attn(q, p1[s:s + hb], ft,
              kvj.reshape(hb * _K, _N, 2 * _MID),
              btj.reshape(hb * _K, _N, 2 * _MID),
              Wd1, bd1.reshape(1, _MID), Wd2, bd2.reshape(1, _MID),
              Wg1, bg1.reshape(1, _MID), Wg2, bg2.reshape(1, _MID),
              W_lin2, b_lin2.reshape(_COUT, 1))
        for s, (ft, q, kv, bt, idxg), (kvj, btj) in zip((0, hb), proj, gath)
    ]
    return (jnp.concatenate(ys, axis=0), p1)


# speedup vs baseline: 1.0444x; 1.0257x over previous
import functools

import jax
import jax.numpy as jnp
from jax import lax
from jax.experimental import pallas as pl
from jax.experimental.pallas import tpu as pltpu
from jax.experimental.pallas import tpu_sc as plsc

_B, _N, _CIN, _COUT, _MID, _K = 8, 2048, 64, 64, 64, 16
_QT = 512
_QA = 256
_NW = 32
_CH = 128
_NCHUNK = (_B * _N * _K) // (_NW * _CH)
_RING = 2


def _proj_knn_body(x_ref, p1_ref, p1t_ref, wpwc_ref, wl1_ref, b1_ref,
                   wq_ref, bq_ref, wk_ref, bk_ref, wv_ref, bv_ref, wd1_ref,
                   ft_ref, q_ref, kv_ref, bt_ref, idx_ref):
    b = pl.program_id(0)
    xb = x_ref[0]
    f = jnp.dot(wpwc_ref[...], xb, preferred_element_type=jnp.float32)
    ft_ref[0] = f
    h = jnp.dot(wl1_ref[...], f, preferred_element_type=jnp.float32) + b1_ref[...]
    qrows = lax.dot_general(h, wq_ref[...], (((0,), (1,)), ((), ())),
                            preferred_element_type=jnp.float32) + bq_ref[...]
    krows = lax.dot_general(h, wk_ref[...], (((0,), (1,)), ((), ())),
                            preferred_element_type=jnp.float32) + bk_ref[...]
    vrows = lax.dot_general(h, wv_ref[...], (((0,), (1,)), ((), ())),
                            preferred_element_type=jnp.float32) + bv_ref[...]
    q_ref[0] = qrows
    kv_ref[0] = jnp.concatenate([krows, vrows], axis=1)
    bt = lax.dot_general(p1_ref[0], wd1_ref[...], (((1,), (1,)), ((), ())),
                         preferred_element_type=jnp.float32)
    bt_ref[0] = jnp.concatenate([bt, jnp.zeros_like(bt)], axis=1)

    qx = p1_ref[0, :, 0:1]
    qy = p1_ref[0, :, 1:2]
    qz = p1_ref[0, :, 2:3]
    px = p1t_ref[0, 0:1, :]
    py = p1t_ref[0, 1:2, :]
    pz = p1t_ref[0, 2:3, :]
    d2 = (qx - px) ** 2 + (qy - py) ** 2 + (qz - pz) ** 2
    col = lax.broadcasted_iota(jnp.int32, (_QT, _N), 1)
    base = b * _N
    cols = []
    for _ in range(_K):
        am = jnp.argmin(d2, axis=1).astype(jnp.int32)[:, None]
        cols.append(am)
        d2 = jnp.where(col == am, jnp.float32(1e30), d2)
    idx_ref[0] = jnp.concatenate(cols, axis=1) + base


def _proj_knn(x, p1, p1t, wpwc, wl1, b1c, wq, bqr, wk, bkr, wv, bvr, wd1):
    nb = x.shape[0]
    nt = _N // _QT
    full = lambda s: pl.BlockSpec(s, lambda b, t: tuple(0 for _ in s))
    return pl.pallas_call(
        _proj_knn_body,
        grid=(nb, nt),
        in_specs=[
            pl.BlockSpec((1, _CIN, _QT), lambda b, t: (b, 0, t)),
            pl.BlockSpec((1, _QT, 3), lambda b, t: (b, t, 0)),
            pl.BlockSpec((1, 3, _N), lambda b, t: (b, 0, 0)),
            full((_COUT, _CIN)),
            full((_MID, _COUT)),
            full((_MID, 1)),
            full((_MID, _MID)), full((1, _MID)),
            full((_MID, _MID)), full((1, _MID)),
            full((_MID, _MID)), full((1, _MID)),
            full((_MID, 3)),
        ],
        out_specs=[
            pl.BlockSpec((1, _COUT, _QT), lambda b, t: (b, 0, t)),
            pl.BlockSpec((1, _QT, _MID), lambda b, t: (b, t, 0)),
            pl.BlockSpec((1, _QT, 2 * _MID), lambda b, t: (b, t, 0)),
            pl.BlockSpec((1, _QT, 2 * _MID), lambda b, t: (b, t, 0)),
            pl.BlockSpec((1, _QT, _K), lambda b, t: (b, t, 0)),
        ],
        out_shape=[
            jax.ShapeDtypeStruct((nb, _COUT, _N), jnp.float32),
            jax.ShapeDtypeStruct((nb, _N, _MID), jnp.float32),
            jax.ShapeDtypeStruct((nb, _N, 2 * _MID), jnp.float32),
            jax.ShapeDtypeStruct((nb, _N, 2 * _MID), jnp.float32),
            jax.ShapeDtypeStruct((nb, _N, _K), jnp.int32),
        ],
    )(x, p1, p1t, wpwc, wl1, b1c, wq, bqr, wk, bkr, wv, bvr, wd1)


def _sc_gather_body(nchunk, idx_hbm, kv_hbm, bt_hbm, kvj_hbm, btj_hbm,
                    idx_v, bk0, bk1, bb0, bb1, sg0, sg1, sw0, sw1):
    wid = lax.axis_index("s") * 2 + lax.axis_index("c")
    bks = [bk0, bk1]
    bbs = [bb0, bb1]
    sgs = [sg0, sg1]
    sws = [sw0, sw1]
    rows_per_w = nchunk * _CH

    pltpu.sync_copy(idx_hbm.at[wid], idx_v)

    def fire_gather(c, r):
        pltpu.async_copy(kv_hbm.at[idx_v.at[c]], bks[r], sgs[r])
        pltpu.async_copy(bt_hbm.at[idx_v.at[c]], bbs[r], sgs[r])

    def wait_gather(r):
        pltpu.make_async_copy(kv_hbm.at[pl.ds(0, _CH)], bks[r], sgs[r]).wait()
        pltpu.make_async_copy(bt_hbm.at[pl.ds(0, _CH)], bbs[r], sgs[r]).wait()

    def fire_write(c, r):
        row0 = wid * rows_per_w + c * _CH
        pltpu.async_copy(bks[r], kvj_hbm.at[pl.ds(row0, _CH)], sws[r])
        pltpu.async_copy(bbs[r], btj_hbm.at[pl.ds(row0, _CH)], sws[r])

    def wait_write(r):
        pltpu.make_async_copy(kvj_hbm.at[pl.ds(0, _CH)], bks[r], sws[r]).wait()
        pltpu.make_async_copy(btj_hbm.at[pl.ds(0, _CH)], bbs[r], sws[r]).wait()

    for r in range(_RING):
        fire_gather(r, r)

    def group(g, carry):
        for r in range(_RING):
            c = g * _RING + r
            wait_gather(r)
            fire_write(c, r)
        for r in range(_RING):
            c2 = g * _RING + r + _RING

            @pl.when(c2 < nchunk)
            def _():
                wait_write(r)
                fire_gather(c2, r)
        return carry

    lax.fori_loop(0, nchunk // _RING, group, 0)
    for r in range(_RING):
        wait_write(r)


def _sc_gather(idx_r, kvf, btf):
    g = kvf.shape[0] * _K
    nchunk = g // (_NW * _CH)
    mesh = plsc.VectorSubcoreMesh(core_axis_name="c", subcore_axis_name="s")
    kfn = functools.partial(
        pl.kernel,
        mesh=mesh,
        compiler_params=pltpu.CompilerParams(use_tc_tiling_on_sc=True),
        out_type=[
            jax.ShapeDtypeStruct((g, 2 * _MID), jnp.float32),
            jax.ShapeDtypeStruct((g, 2 * _MID), jnp.float32),
        ],
        scratch_types=(
            [pltpu.VMEM((nchunk, _CH), jnp.int32)]
            + [pltpu.VMEM((_CH, 2 * _MID), jnp.float32) for _ in range(2 * _RING)]
            + [pltpu.SemaphoreType.DMA for _ in range(2 * _RING)]
        ),
    )(functools.partial(_sc_gather_body, nchunk))
    return kfn(idx_r, kvf, btf)


def _attn_body(q_ref, p1_ref, ft_ref, kvj_ref, btj_ref,
               wd1_ref, bd1_ref, wd2_ref, bd2_ref,
               wg1_ref, bg1_ref, wg2_ref, bg2_ref,
               wl2_ref, bl2_ref, y_ref):
    qk = _QA * _K
    p1b = p1_ref[0]
    a_q = lax.dot_general(p1b, wd1_ref[...], (((1,), (1,)), ((), ())),
                          preferred_element_type=jnp.float32) + bd1_ref[...]
    rel1 = (jnp.reshape(a_q, (1, _QA, _MID))
            - btj_ref[:, :, 0:_MID])
    relu1 = jnp.maximum(rel1, 0.0)
    pos = lax.dot_general(jnp.reshape(relu1, (qk, _MID)), wd2_ref[...],
                          (((1,), (1,)), ((), ())),
                          preferred_element_type=jnp.float32) + bd2_ref[...]
    pos3 = jnp.reshape(pos, (_K, _QA, _MID))

    e3 = (jnp.reshape(q_ref[0], (1, _QA, _MID))
          - kvj_ref[:, :, 0:_MID]
          + pos3)
    t1 = jnp.maximum(
        lax.dot_general(jnp.reshape(e3, (qk, _MID)), wg1_ref[...],
                        (((1,), (1,)), ((), ())),
                        preferred_element_type=jnp.float32) + bg1_ref[...], 0.0)
    logits = lax.dot_general(t1, wg2_ref[...], (((1,), (1,)), ((), ())),
                             preferred_element_type=jnp.float32) + bg2_ref[...]
    l3 = jnp.reshape(logits, (_K, _QA, _MID))
    mx = jnp.max(l3, axis=0, keepdims=True)
    ex = jnp.exp(l3 - mx)
    attn = ex * (1.0 / jnp.sum(ex, axis=0, keepdims=True))

    vp = kvj_ref[:, :, _MID:2 * _MID] + pos3
    agg = jnp.sum(attn * vp, axis=0)
    yt = lax.dot_general(wl2_ref[...], agg, (((1,), (1,)), ((), ())),
                         preferred_element_type=jnp.float32)
    y_ref[0] = yt + bl2_ref[...] + ft_ref[0]


def _attn(q, p1, ft, kvj, btj, wd1, bd1r, wd2, bd2r, wg1, bg1r, wg2, bg2r,
          wl2, bl2c):
    nb = q.shape[0]
    nt = _N // _QA
    full = lambda s: pl.BlockSpec(s, lambda b, t: tuple(0 for _ in s))
    return pl.pallas_call(
        _attn_body,
        grid=(nb, nt),
        in_specs=[
            pl.BlockSpec((1, _QA, _MID), lambda b, t: (b, t, 0)),
            pl.BlockSpec((1, _QA, 3), lambda b, t: (b, t, 0)),
            pl.BlockSpec((1, _COUT, _QA), lambda b, t: (b, 0, t)),
            pl.BlockSpec((_K, _QA, 2 * _MID), lambda b, t: (b, t, 0)),
            pl.BlockSpec((_K, _QA, 2 * _MID), lambda b, t: (b, t, 0)),
            full((_MID, 3)), full((1, _MID)),
            full((_MID, _MID)), full((1, _MID)),
            full((_MID, _MID)), full((1, _MID)),
            full((_MID, _MID)), full((1, _MID)),
            full((_COUT, _MID)), full((_COUT, 1)),
        ],
        out_specs=pl.BlockSpec((1, _COUT, _QA), lambda b, t: (b, 0, t)),
        out_shape=jax.ShapeDtypeStruct((nb, _COUT, _N), jnp.float32),
    )(q, p1, ft, kvj, btj, wd1, bd1r, wd2, bd2r, wg1, bg1r, wg2, bg2r,
      wl2, bl2c)


def kernel(x, p1, W_pwc, W_lin1, b_lin1, Wq, bq, Wk, bk, Wv, bv,
           Wd1, bd1, Wd2, bd2, Wg1, bg1, Wg2, bg2, W_lin2, b_lin2):
    p1t = jnp.transpose(p1, (0, 2, 1))
    hb = _B // 4
    starts = tuple(range(0, _B, hb))

    proj = [
        _proj_knn(
            x[s:s + hb], p1[s:s + hb], p1t[s:s + hb],
            W_pwc, W_lin1, b_lin1.reshape(_MID, 1),
            Wq, bq.reshape(1, _MID), Wk, bk.reshape(1, _MID),
            Wv, bv.reshape(1, _MID), Wd1)
        for s in starts
    ]
    gath = [
        _sc_gather(jnp.transpose(idxg, (0, 2, 1)).reshape(_NW, -1, _CH),
                   kv.reshape(hb * _N, 2 * _MID),
                   bt.reshape(hb * _N, 2 * _MID))
        for (ft, q, kv, bt, idxg) in proj
    ]
    ys = [
        _attn(q, p1[s:s + hb], ft,
              kvj.reshape(hb * _K, _N, 2 * _MID),
              btj.reshape(hb * _K, _N, 2 * _MID),
              Wd1, bd1.reshape(1, _MID), Wd2, bd2.reshape(1, _MID),
              Wg1, bg1.reshape(1, _MID), Wg2, bg2.reshape(1, _MID),
              W_lin2, b_lin2.reshape(_COUT, 1))
        for s, (ft, q, kv, bt, idxg), (kvj, btj) in zip(starts, proj, gath)
    ]
    return (jnp.concatenate(ys, axis=0), p1)


# speedup vs baseline: 1.0456x; 1.0011x over previous
import functools

import jax
import jax.numpy as jnp
from jax import lax
from jax.experimental import pallas as pl
from jax.experimental.pallas import tpu as pltpu
from jax.experimental.pallas import tpu_sc as plsc

_B, _N, _CIN, _COUT, _MID, _K = 8, 2048, 64, 64, 64, 16
_QT = 512
_QA = 256
_NW = 32
_CH = 128
_NCHUNK = (_B * _N * _K) // (_NW * _CH)
_RING = 2


def _proj_knn_body(x_ref, p1_ref, p1t_ref, wpwc_ref, wl1_ref, b1_ref,
                   wq_ref, bq_ref, wk_ref, bk_ref, wv_ref, bv_ref, wd1_ref,
                   ft_ref, q_ref, kv_ref, bt_ref, idx_ref):
    b = pl.program_id(0)
    xb = x_ref[0]
    f = jnp.dot(wpwc_ref[...], xb, preferred_element_type=jnp.float32)
    ft_ref[0] = f
    h = jnp.dot(wl1_ref[...], f, preferred_element_type=jnp.float32) + b1_ref[...]
    qrows = lax.dot_general(h, wq_ref[...], (((0,), (1,)), ((), ())),
                            preferred_element_type=jnp.float32) + bq_ref[...]
    krows = lax.dot_general(h, wk_ref[...], (((0,), (1,)), ((), ())),
                            preferred_element_type=jnp.float32) + bk_ref[...]
    vrows = lax.dot_general(h, wv_ref[...], (((0,), (1,)), ((), ())),
                            preferred_element_type=jnp.float32) + bv_ref[...]
    q_ref[0] = qrows
    kv_ref[0] = jnp.concatenate([krows, vrows], axis=1)
    bt = lax.dot_general(p1_ref[0], wd1_ref[...], (((1,), (1,)), ((), ())),
                         preferred_element_type=jnp.float32)
    bt_ref[0] = jnp.concatenate([bt, jnp.zeros_like(bt)], axis=1)

    qx = p1_ref[0, :, 0:1]
    qy = p1_ref[0, :, 1:2]
    qz = p1_ref[0, :, 2:3]
    px = p1t_ref[0, 0:1, :]
    py = p1t_ref[0, 1:2, :]
    pz = p1t_ref[0, 2:3, :]
    d2 = (qx - px) ** 2 + (qy - py) ** 2 + (qz - pz) ** 2
    col = lax.broadcasted_iota(jnp.int32, (_QT, _N), 1)
    base = b * _N
    cols = []
    for _ in range(_K):
        am = jnp.argmin(d2, axis=1).astype(jnp.int32)[:, None]
        cols.append(am)
        d2 = jnp.where(col == am, jnp.float32(1e30), d2)
    idx_ref[0] = jnp.concatenate(cols, axis=1) + base


def _proj_knn(x, p1, p1t, wpwc, wl1, b1c, wq, bqr, wk, bkr, wv, bvr, wd1):
    nb = x.shape[0]
    nt = _N // _QT
    full = lambda s: pl.BlockSpec(s, lambda b, t: tuple(0 for _ in s))
    return pl.pallas_call(
        _proj_knn_body,
        grid=(nb, nt),
        in_specs=[
            pl.BlockSpec((1, _CIN, _QT), lambda b, t: (b, 0, t)),
            pl.BlockSpec((1, _QT, 3), lambda b, t: (b, t, 0)),
            pl.BlockSpec((1, 3, _N), lambda b, t: (b, 0, 0)),
            full((_COUT, _CIN)),
            full((_MID, _COUT)),
            full((_MID, 1)),
            full((_MID, _MID)), full((1, _MID)),
            full((_MID, _MID)), full((1, _MID)),
            full((_MID, _MID)), full((1, _MID)),
            full((_MID, 3)),
        ],
        out_specs=[
            pl.BlockSpec((1, _COUT, _QT), lambda b, t: (b, 0, t)),
            pl.BlockSpec((1, _QT, _MID), lambda b, t: (b, t, 0)),
            pl.BlockSpec((1, _QT, 2 * _MID), lambda b, t: (b, t, 0)),
            pl.BlockSpec((1, _QT, 2 * _MID), lambda b, t: (b, t, 0)),
            pl.BlockSpec((1, _QT, _K), lambda b, t: (b, t, 0)),
        ],
        out_shape=[
            jax.ShapeDtypeStruct((nb, _COUT, _N), jnp.float32),
            jax.ShapeDtypeStruct((nb, _N, _MID), jnp.float32),
            jax.ShapeDtypeStruct((nb, _N, 2 * _MID), jnp.float32),
            jax.ShapeDtypeStruct((nb, _N, 2 * _MID), jnp.float32),
            jax.ShapeDtypeStruct((nb, _N, _K), jnp.int32),
        ],
    )(x, p1, p1t, wpwc, wl1, b1c, wq, bqr, wk, bkr, wv, bvr, wd1)


def _sc_gather_body(nchunk, idx_hbm, kv_hbm, bt_hbm, kvj_hbm, btj_hbm,
                    idx_v, bk0, bk1, bb0, bb1, sg0, sg1, sw0, sw1):
    wid = lax.axis_index("s") * 2 + lax.axis_index("c")
    bks = [bk0, bk1]
    bbs = [bb0, bb1]
    sgs = [sg0, sg1]
    sws = [sw0, sw1]
    rows_per_w = nchunk * _CH

    pltpu.sync_copy(idx_hbm.at[wid], idx_v)

    def fire_gather(c, r):
        pltpu.async_copy(kv_hbm.at[idx_v.at[c]], bks[r], sgs[r])
        pltpu.async_copy(bt_hbm.at[idx_v.at[c]], bbs[r], sgs[r])

    def wait_gather(r):
        pltpu.make_async_copy(kv_hbm.at[pl.ds(0, _CH)], bks[r], sgs[r]).wait()
        pltpu.make_async_copy(bt_hbm.at[pl.ds(0, _CH)], bbs[r], sgs[r]).wait()

    def fire_write(c, r):
        row0 = wid * rows_per_w + c * _CH
        pltpu.async_copy(bks[r], kvj_hbm.at[pl.ds(row0, _CH)], sws[r])
        pltpu.async_copy(bbs[r], btj_hbm.at[pl.ds(row0, _CH)], sws[r])

    def wait_write(r):
        pltpu.make_async_copy(kvj_hbm.at[pl.ds(0, _CH)], bks[r], sws[r]).wait()
        pltpu.make_async_copy(btj_hbm.at[pl.ds(0, _CH)], bbs[r], sws[r]).wait()

    for r in range(_RING):
        fire_gather(r, r)

    def group(g, carry):
        for r in range(_RING):
            c = g * _RING + r
            wait_gather(r)
            fire_write(c, r)
        for r in range(_RING):
            c2 = g * _RING + r + _RING

            @pl.when(c2 < nchunk)
            def _():
                wait_write(r)
                fire_gather(c2, r)
        return carry

    lax.fori_loop(0, nchunk // _RING, group, 0)
    for r in range(_RING):
        wait_write(r)


def _sc_gather(idx_r, kvf, btf):
    g = kvf.shape[0] * _K
    nchunk = g // (_NW * _CH)
    mesh = plsc.VectorSubcoreMesh(core_axis_name="c", subcore_axis_name="s")
    kfn = functools.partial(
        pl.kernel,
        mesh=mesh,
        compiler_params=pltpu.CompilerParams(use_tc_tiling_on_sc=True),
        out_type=[
            jax.ShapeDtypeStruct((g, 2 * _MID), jnp.float32),
            jax.ShapeDtypeStruct((g, 2 * _MID), jnp.float32),
        ],
        scratch_types=(
            [pltpu.VMEM((nchunk, _CH), jnp.int32)]
            + [pltpu.VMEM((_CH, 2 * _MID), jnp.float32) for _ in range(2 * _RING)]
            + [pltpu.SemaphoreType.DMA for _ in range(2 * _RING)]
        ),
    )(functools.partial(_sc_gather_body, nchunk))
    return kfn(idx_r, kvf, btf)


def _attn_body(q_ref, p1_ref, ft_ref, kvj_ref, btj_ref,
               wd1_ref, bd1_ref, wd2_ref, bd2_ref,
               wg1_ref, bg1_ref, wg2_ref, bg2_ref,
               wl2_ref, bl2_ref, y_ref):
    qk = _QA * _K
    p1b = p1_ref[0]
    a_q = lax.dot_general(p1b, wd1_ref[...], (((1,), (1,)), ((), ())),
                          preferred_element_type=jnp.float32) + bd1_ref[...]
    rel1 = (jnp.reshape(a_q, (1, _QA, _MID))
            - btj_ref[:, :, 0:_MID])
    relu1 = jnp.maximum(rel1, 0.0)
    pos = lax.dot_general(jnp.reshape(relu1, (qk, _MID)), wd2_ref[...],
                          (((1,), (1,)), ((), ())),
                          preferred_element_type=jnp.float32) + bd2_ref[...]
    pos3 = jnp.reshape(pos, (_K, _QA, _MID))

    e3 = (jnp.reshape(q_ref[0], (1, _QA, _MID))
          - kvj_ref[:, :, 0:_MID]
          + pos3)
    t1 = jnp.maximum(
        lax.dot_general(jnp.reshape(e3, (qk, _MID)), wg1_ref[...],
                        (((1,), (1,)), ((), ())),
                        preferred_element_type=jnp.float32) + bg1_ref[...], 0.0)
    logits = lax.dot_general(t1, wg2_ref[...], (((1,), (1,)), ((), ())),
                             preferred_element_type=jnp.float32) + bg2_ref[...]
    l3 = jnp.reshape(logits, (_K, _QA, _MID))
    mx = jnp.max(l3, axis=0, keepdims=True)
    ex = jnp.exp(l3 - mx)
    attn = ex * (1.0 / jnp.sum(ex, axis=0, keepdims=True))

    vp = kvj_ref[:, :, _MID:2 * _MID] + pos3
    agg = jnp.sum(attn * vp, axis=0)
    yt = lax.dot_general(wl2_ref[...], agg, (((1,), (1,)), ((), ())),
                         preferred_element_type=jnp.float32)
    y_ref[0] = yt + bl2_ref[...] + ft_ref[0]


def _attn(q, p1, ft, kvj, btj, wd1, bd1r, wd2, bd2r, wg1, bg1r, wg2, bg2r,
          wl2, bl2c):
    nb = q.shape[0]
    nt = _N // _QA
    full = lambda s: pl.BlockSpec(s, lambda b, t: tuple(0 for _ in s))
    return pl.pallas_call(
        _attn_body,
        grid=(nb, nt),
        in_specs=[
            pl.BlockSpec((1, _QA, _MID), lambda b, t: (b, t, 0)),
            pl.BlockSpec((1, _QA, 3), lambda b, t: (b, t, 0)),
            pl.BlockSpec((1, _COUT, _QA), lambda b, t: (b, 0, t)),
            pl.BlockSpec((_K, _QA, 2 * _MID), lambda b, t: (b, t, 0)),
            pl.BlockSpec((_K, _QA, 2 * _MID), lambda b, t: (b, t, 0)),
            full((_MID, 3)), full((1, _MID)),
            full((_MID, _MID)), full((1, _MID)),
            full((_MID, _MID)), full((1, _MID)),
            full((_MID, _MID)), full((1, _MID)),
            full((_COUT, _MID)), full((_COUT, 1)),
        ],
        out_specs=pl.BlockSpec((1, _COUT, _QA), lambda b, t: (b, 0, t)),
        out_shape=jax.ShapeDtypeStruct((nb, _COUT, _N), jnp.float32),
    )(q, p1, ft, kvj, btj, wd1, bd1r, wd2, bd2r, wg1, bg1r, wg2, bg2r,
      wl2, bl2c)


def kernel(x, p1, W_pwc, W_lin1, b_lin1, Wq, bq, Wk, bk, Wv, bv,
           Wd1, bd1, Wd2, bd2, Wg1, bg1, Wg2, bg2, W_lin2, b_lin2):
    p1t = jnp.transpose(p1, (0, 2, 1))
    hb = _B // 8
    starts = tuple(range(0, _B, hb))

    proj = [
        _proj_knn(
            x[s:s + hb], p1[s:s + hb], p1t[s:s + hb],
            W_pwc, W_lin1, b_lin1.reshape(_MID, 1),
            Wq, bq.reshape(1, _MID), Wk, bk.reshape(1, _MID),
            Wv, bv.reshape(1, _MID), Wd1)
        for s in starts
    ]
    gath = [
        _sc_gather(jnp.transpose(idxg, (0, 2, 1)).reshape(_NW, -1, _CH),
                   kv.reshape(hb * _N, 2 * _MID),
                   bt.reshape(hb * _N, 2 * _MID))
        for (ft, q, kv, bt, idxg) in proj
    ]
    ys = [
        _attn(q, p1[s:s + hb], ft,
              kvj.reshape(hb * _K, _N, 2 * _MID),
              btj.reshape(hb * _K, _N, 2 * _MID),
              Wd1, bd1.reshape(1, _MID), Wd2, bd2.reshape(1, _MID),
              Wg1, bg1.reshape(1, _MID), Wg2, bg2.reshape(1, _MID),
              W_lin2, b_lin2.reshape(_COUT, 1))
        for s, (ft, q, kv, bt, idxg), (kvj, btj) in zip(starts, proj, gath)
    ]
    return (jnp.concatenate(ys, axis=0), p1)
